# Initial kernel scaffold; baseline (speedup 1.0000x reference)
import jax
import jax.numpy as jnp
from jax.experimental import pallas as pl


def kernel(words, masks, pos, ner, deprel, head, subj_pos, obj_pos, subj_type, obj_type, pos_table, ner_table):
    raise NotImplementedError("write your pallas kernel here")


# speedup vs baseline: 1.6959x; 1.6959x over previous
import jax
import jax.numpy as jnp
from jax.experimental import pallas as pl

B = 1024
S = 200
N_POS = 53
N_NER = 25
POS_DIM = 30
NER_DIM = 30

_BS = 8


def _body(masks_ref, pos_ref, ner_ref, head_ref, pt_ref, nt_ref,
          pos_out, ner_out, dep_out, pad_out, seq_out, adj_out):
    bs = masks_ref.shape[0]
    l = jnp.sum((masks_ref[...] == 0.0).astype(jnp.int32), axis=1, keepdims=True)
    l3 = l[:, :, None]
    head = head_ref[...][:, None, :]
    rows = jax.lax.broadcasted_iota(jnp.int32, (bs, S, S), 1)
    cols = jax.lax.broadcasted_iota(jnp.int32, (bs, S, S), 2)
    col_valid = cols < l3
    adj_b = (head > 0) & (head - 1 == rows) & col_valid
    adj_out[...] = adj_b.astype(jnp.float32)
    dep_out[...] = jnp.logical_not(adj_b)
    pad_out[...] = jnp.logical_not(col_valid)
    seq_out[...] = cols > rows

    pos_oh = (pos_ref[...][:, :, None] ==
              jax.lax.broadcasted_iota(jnp.int32, (bs, S, N_POS), 2)
              ).astype(jnp.float32).reshape(bs * S, N_POS)
    pos_out[...] = jnp.dot(pos_oh, pt_ref[...],
                           preferred_element_type=jnp.float32).reshape(bs, S, POS_DIM)
    ner_oh = (ner_ref[...][:, :, None] ==
              jax.lax.broadcasted_iota(jnp.int32, (bs, S, N_NER), 2)
              ).astype(jnp.float32).reshape(bs * S, N_NER)
    ner_out[...] = jnp.dot(ner_oh, nt_ref[...],
                           preferred_element_type=jnp.float32).reshape(bs, S, NER_DIM)


def kernel(words, masks, pos, ner, deprel, head, subj_pos, obj_pos, subj_type, obj_type,
           pos_table, ner_table):
    del words, deprel, subj_pos, obj_pos, subj_type, obj_type
    grid = (B // _BS,)
    in_specs = [
        pl.BlockSpec((_BS, S), lambda i: (i, 0)),
        pl.BlockSpec((_BS, S), lambda i: (i, 0)),
        pl.BlockSpec((_BS, S), lambda i: (i, 0)),
        pl.BlockSpec((_BS, S), lambda i: (i, 0)),
        pl.BlockSpec((N_POS, POS_DIM), lambda i: (0, 0)),
        pl.BlockSpec((N_NER, NER_DIM), lambda i: (0, 0)),
    ]
    out_specs = [
        pl.BlockSpec((_BS, S, POS_DIM), lambda i: (i, 0, 0)),
        pl.BlockSpec((_BS, S, NER_DIM), lambda i: (i, 0, 0)),
        pl.BlockSpec((_BS, S, S), lambda i: (i, 0, 0)),
        pl.BlockSpec((_BS, S, S), lambda i: (i, 0, 0)),
        pl.BlockSpec((_BS, S, S), lambda i: (i, 0, 0)),
        pl.BlockSpec((_BS, S, S), lambda i: (i, 0, 0)),
    ]
    out_shape = [
        jax.ShapeDtypeStruct((B, S, POS_DIM), jnp.float32),
        jax.ShapeDtypeStruct((B, S, NER_DIM), jnp.float32),
        jax.ShapeDtypeStruct((B, S, S), jnp.bool_),
        jax.ShapeDtypeStruct((B, S, S), jnp.bool_),
        jax.ShapeDtypeStruct((B, S, S), jnp.bool_),
        jax.ShapeDtypeStruct((B, S, S), jnp.float32),
    ]
    pos_embs, ner_embs, dep_mask, pad_mask, seq_mask, adj = pl.pallas_call(
        _body,
        grid=grid,
        in_specs=in_specs,
        out_specs=out_specs,
        out_shape=out_shape,
    )(masks, pos, ner, head, pos_table, ner_table)
    return (pos_embs, ner_embs, dep_mask, pad_mask, seq_mask, adj)


# speedup vs baseline: 1.6973x; 1.0008x over previous
import jax
import jax.numpy as jnp
from jax.experimental import pallas as pl

B = 1024
S = 200
N_POS = 53
N_NER = 25
POS_DIM = 30
NER_DIM = 30

_BS = 16


def _body(masks_ref, pos_ref, ner_ref, head_ref, pt_ref, nt_ref,
          pos_out, ner_out, dep_out, pad_out, seq_out, adj_out):
    bs = masks_ref.shape[0]
    l = jnp.sum((masks_ref[...] == 0.0).astype(jnp.int32), axis=1, keepdims=True)
    l3 = l[:, :, None]
    head = head_ref[...][:, None, :]
    rows = jax.lax.broadcasted_iota(jnp.int32, (bs, S, S), 1)
    cols = jax.lax.broadcasted_iota(jnp.int32, (bs, S, S), 2)
    col_valid = cols < l3
    adj_b = (head > 0) & (head - 1 == rows) & col_valid
    adj_out[...] = adj_b.astype(jnp.float32)
    dep_out[...] = jnp.logical_not(adj_b)
    pad_out[...] = jnp.logical_not(col_valid)
    seq_out[...] = cols > rows

    pos_oh = (pos_ref[...][:, :, None] ==
              jax.lax.broadcasted_iota(jnp.int32, (bs, S, N_POS), 2)
              ).astype(jnp.float32).reshape(bs * S, N_POS)
    pos_out[...] = jnp.dot(pos_oh, pt_ref[...],
                           preferred_element_type=jnp.float32).reshape(bs, S, POS_DIM)
    ner_oh = (ner_ref[...][:, :, None] ==
              jax.lax.broadcasted_iota(jnp.int32, (bs, S, N_NER), 2)
              ).astype(jnp.float32).reshape(bs * S, N_NER)
    ner_out[...] = jnp.dot(ner_oh, nt_ref[...],
                           preferred_element_type=jnp.float32).reshape(bs, S, NER_DIM)


def kernel(words, masks, pos, ner, deprel, head, subj_pos, obj_pos, subj_type, obj_type,
           pos_table, ner_table):
    del words, deprel, subj_pos, obj_pos, subj_type, obj_type
    grid = (B // _BS,)
    in_specs = [
        pl.BlockSpec((_BS, S), lambda i: (i, 0)),
        pl.BlockSpec((_BS, S), lambda i: (i, 0)),
        pl.BlockSpec((_BS, S), lambda i: (i, 0)),
        pl.BlockSpec((_BS, S), lambda i: (i, 0)),
        pl.BlockSpec((N_POS, POS_DIM), lambda i: (0, 0)),
        pl.BlockSpec((N_NER, NER_DIM), lambda i: (0, 0)),
    ]
    out_specs = [
        pl.BlockSpec((_BS, S, POS_DIM), lambda i: (i, 0, 0)),
        pl.BlockSpec((_BS, S, NER_DIM), lambda i: (i, 0, 0)),
        pl.BlockSpec((_BS, S, S), lambda i: (i, 0, 0)),
        pl.BlockSpec((_BS, S, S), lambda i: (i, 0, 0)),
        pl.BlockSpec((_BS, S, S), lambda i: (i, 0, 0)),
        pl.BlockSpec((_BS, S, S), lambda i: (i, 0, 0)),
    ]
    out_shape = [
        jax.ShapeDtypeStruct((B, S, POS_DIM), jnp.float32),
        jax.ShapeDtypeStruct((B, S, NER_DIM), jnp.float32),
        jax.ShapeDtypeStruct((B, S, S), jnp.bool_),
        jax.ShapeDtypeStruct((B, S, S), jnp.bool_),
        jax.ShapeDtypeStruct((B, S, S), jnp.bool_),
        jax.ShapeDtypeStruct((B, S, S), jnp.float32),
    ]
    pos_embs, ner_embs, dep_mask, pad_mask, seq_mask, adj = pl.pallas_call(
        _body,
        grid=grid,
        in_specs=in_specs,
        out_specs=out_specs,
        out_shape=out_shape,
    )(masks, pos, ner, head, pos_table, ner_table)
    return (pos_embs, ner_embs, dep_mask, pad_mask, seq_mask, adj)


# speedup vs baseline: 4.3192x; 2.5447x over previous
import jax
import jax.numpy as jnp
from jax.experimental import pallas as pl

B = 1024
S = 200
N_POS = 53
N_NER = 25
POS_DIM = 30
NER_DIM = 30

_RBLK = 8
_BS = 8


def _mask_body(masks_ref, head_ref, adj_ref, dep_ref, pad_ref, seq_ref):
    i = pl.program_id(0)
    l = jnp.sum((masks_ref[...] == 0.0).astype(jnp.int32), axis=0, keepdims=True)
    head2 = head_ref[...]
    cvec2 = jax.lax.broadcasted_iota(jnp.int32, (S, 1), 0)
    col_valid2 = cvec2 < l
    head_eff = jnp.where((head2 > 0) & col_valid2, head2, 0)
    rvec = jax.lax.broadcasted_iota(jnp.int32, (_RBLK, 1, 1), 0) + i * _RBLK
    eq = head_eff[None, :, :] == rvec + 1
    adj_ref[...] = eq.astype(jnp.float32)
    dep_ref[...] = jnp.where(eq, 0, 1).astype(jnp.int8)
    pad2 = jnp.where(col_valid2, 0, 1)
    pad_ref[...] = jnp.broadcast_to(pad2[None, :, :],
                                    (_RBLK, S, B)).astype(jnp.int8)
    cols3 = jax.lax.broadcasted_iota(jnp.int32, (_RBLK, S, 1), 1)
    seq3 = jnp.where(cols3 > rvec, 1, 0)
    seq_ref[...] = jnp.broadcast_to(seq3, (_RBLK, S, B)).astype(jnp.int8)


def _emb_body(pos_ref, ner_ref, pt_ref, nt_ref, pos_out, ner_out):
    bs = pos_ref.shape[0]
    pos_oh = (pos_ref[...][:, :, None] ==
              jax.lax.broadcasted_iota(jnp.int32, (bs, S, N_POS), 2)
              ).astype(jnp.float32).reshape(bs * S, N_POS)
    pos_out[...] = jnp.dot(pos_oh, pt_ref[...],
                           preferred_element_type=jnp.float32).reshape(bs, S, POS_DIM)
    ner_oh = (ner_ref[...][:, :, None] ==
              jax.lax.broadcasted_iota(jnp.int32, (bs, S, N_NER), 2)
              ).astype(jnp.float32).reshape(bs * S, N_NER)
    ner_out[...] = jnp.dot(ner_oh, nt_ref[...],
                           preferred_element_type=jnp.float32).reshape(bs, S, NER_DIM)


def kernel(words, masks, pos, ner, deprel, head, subj_pos, obj_pos, subj_type, obj_type,
           pos_table, ner_table):
    del words, deprel, subj_pos, obj_pos, subj_type, obj_type
    masks_t = masks.T
    head_t = head.T

    adj_t, dep_t, pad_t, seq_t = pl.pallas_call(
        _mask_body,
        grid=(S // _RBLK,),
        in_specs=[
            pl.BlockSpec((S, B), lambda i: (0, 0)),
            pl.BlockSpec((S, B), lambda i: (0, 0)),
        ],
        out_specs=[pl.BlockSpec((_RBLK, S, B), lambda i: (i, 0, 0))] * 4,
        out_shape=[
            jax.ShapeDtypeStruct((S, S, B), jnp.float32),
            jax.ShapeDtypeStruct((S, S, B), jnp.int8),
            jax.ShapeDtypeStruct((S, S, B), jnp.int8),
            jax.ShapeDtypeStruct((S, S, B), jnp.int8),
        ],
    )(masks_t, head_t)

    adj = jnp.transpose(adj_t, (2, 0, 1))
    dep_mask = jnp.transpose(dep_t, (2, 0, 1)).view(jnp.bool_)
    pad_mask = jnp.transpose(pad_t, (2, 0, 1)).view(jnp.bool_)
    seq_mask = jnp.transpose(seq_t, (2, 0, 1)).view(jnp.bool_)

    pos_embs, ner_embs = pl.pallas_call(
        _emb_body,
        grid=(B // _BS,),
        in_specs=[
            pl.BlockSpec((_BS, S), lambda i: (i, 0)),
            pl.BlockSpec((_BS, S), lambda i: (i, 0)),
            pl.BlockSpec((N_POS, POS_DIM), lambda i: (0, 0)),
            pl.BlockSpec((N_NER, NER_DIM), lambda i: (0, 0)),
        ],
        out_specs=[
            pl.BlockSpec((_BS, S, POS_DIM), lambda i: (i, 0, 0)),
            pl.BlockSpec((_BS, S, NER_DIM), lambda i: (i, 0, 0)),
        ],
        out_shape=[
            jax.ShapeDtypeStruct((B, S, POS_DIM), jnp.float32),
            jax.ShapeDtypeStruct((B, S, NER_DIM), jnp.float32),
        ],
    )(pos, ner, pos_table, ner_table)

    return (pos_embs, ner_embs, dep_mask, pad_mask, seq_mask, adj)


# speedup vs baseline: 12.4293x; 2.8777x over previous
import jax
import jax.numpy as jnp
from jax.experimental import pallas as pl

B = 1024
S = 200
N_POS = 53
N_NER = 25
POS_DIM = 30
NER_DIM = 30

_RBLK = 8
_SBLK = 8


def _mask_body(masks_ref, head_ref, adj_ref, dep_ref):
    i = pl.program_id(0)
    l = jnp.sum((masks_ref[...] == 0.0).astype(jnp.int32), axis=0, keepdims=True)
    head2 = head_ref[...]
    cvec2 = jax.lax.broadcasted_iota(jnp.int32, (S, 1), 0)
    col_valid2 = cvec2 < l
    head_eff = jnp.where((head2 > 0) & col_valid2, head2, 0)
    rvec = jax.lax.broadcasted_iota(jnp.int32, (_RBLK, 1, 1), 0) + i * _RBLK
    eq = head_eff[None, :, :] == rvec + 1
    adj_ref[...] = eq.astype(jnp.float32)
    dep_ref[...] = jnp.where(eq, 0, 1).astype(jnp.int8)


def _emb_body(pos_ref, ner_ref, ptt_ref, ntt_ref, pos_out, ner_out):
    ptt = ptt_ref[...]
    ntt = ntt_ref[...]
    kp = jax.lax.broadcasted_iota(jnp.int32, (N_POS, 1), 0)
    kn = jax.lax.broadcasted_iota(jnp.int32, (N_NER, 1), 0)
    for s in range(_SBLK):
        prow = pos_ref[s:s + 1, :]
        oh = (kp == prow).astype(jnp.float32)
        res = jnp.dot(ptt, oh, preferred_element_type=jnp.float32)
        pos_out[:, s:s + 1, :] = res[:, None, :]
        nrow = ner_ref[s:s + 1, :]
        ohn = (kn == nrow).astype(jnp.float32)
        resn = jnp.dot(ntt, ohn, preferred_element_type=jnp.float32)
        ner_out[:, s:s + 1, :] = resn[:, None, :]


def kernel(words, masks, pos, ner, deprel, head, subj_pos, obj_pos, subj_type, obj_type,
           pos_table, ner_table):
    del words, deprel, subj_pos, obj_pos, subj_type, obj_type
    masks_t = masks.T
    head_t = head.T
    pos_t = pos.T
    ner_t = ner.T

    adj_t, dep_t = pl.pallas_call(
        _mask_body,
        grid=(S // _RBLK,),
        in_specs=[
            pl.BlockSpec((S, B), lambda i: (0, 0)),
            pl.BlockSpec((S, B), lambda i: (0, 0)),
        ],
        out_specs=[
            pl.BlockSpec((_RBLK, S, B), lambda i: (i, 0, 0)),
            pl.BlockSpec((_RBLK, S, B), lambda i: (i, 0, 0)),
        ],
        out_shape=[
            jax.ShapeDtypeStruct((S, S, B), jnp.float32),
            jax.ShapeDtypeStruct((S, S, B), jnp.int8),
        ],
    )(masks_t, head_t)

    adj = jnp.transpose(adj_t, (2, 0, 1))
    dep_mask = jnp.transpose(dep_t, (2, 0, 1)).view(jnp.bool_)

    pos_et, ner_et = pl.pallas_call(
        _emb_body,
        grid=(S // _SBLK,),
        in_specs=[
            pl.BlockSpec((_SBLK, B), lambda i: (i, 0)),
            pl.BlockSpec((_SBLK, B), lambda i: (i, 0)),
            pl.BlockSpec((POS_DIM, N_POS), lambda i: (0, 0)),
            pl.BlockSpec((NER_DIM, N_NER), lambda i: (0, 0)),
        ],
        out_specs=[
            pl.BlockSpec((POS_DIM, _SBLK, B), lambda i: (0, i, 0)),
            pl.BlockSpec((NER_DIM, _SBLK, B), lambda i: (0, i, 0)),
        ],
        out_shape=[
            jax.ShapeDtypeStruct((POS_DIM, S, B), jnp.float32),
            jax.ShapeDtypeStruct((NER_DIM, S, B), jnp.float32),
        ],
    )(pos_t, ner_t, pos_table.T, ner_table.T)

    pos_embs = jnp.transpose(pos_et, (2, 1, 0))
    ner_embs = jnp.transpose(ner_et, (2, 1, 0))

    l = jnp.sum((masks == 0.0).astype(jnp.int32), axis=1)
    alen = jnp.arange(S)
    amask = alen[None, :] < l[:, None]
    pad_mask = jnp.broadcast_to((~amask)[:, None, :], (B, S, S))
    seq_mask = jnp.broadcast_to(~(alen[None, None, :] <= alen[None, :, None]),
                                (B, S, S))

    return (pos_embs, ner_embs, dep_mask, pad_mask, seq_mask, adj)


# speedup vs baseline: 14.1224x; 1.1362x over previous
import jax
import jax.numpy as jnp
from jax.experimental import pallas as pl

B = 1024
S = 200
N_POS = 53
N_NER = 25
POS_DIM = 30
NER_DIM = 30

_RBLK = 8
_SBLK = 8


def _mask_body(masks_ref, head_ref, adj_ref):
    i = pl.program_id(0)
    l = jnp.sum((masks_ref[...] == 0.0).astype(jnp.int32), axis=0, keepdims=True)
    head2 = head_ref[...]
    cvec2 = jax.lax.broadcasted_iota(jnp.int32, (S, 1), 0)
    col_valid2 = cvec2 < l
    head_eff = jnp.where((head2 > 0) & col_valid2, head2, 0)
    rvec = jax.lax.broadcasted_iota(jnp.int32, (_RBLK, 1, 1), 0) + i * _RBLK
    eq = head_eff[None, :, :] == rvec + 1
    adj_ref[...] = eq.astype(jnp.float32)


def _emb_body(pos_ref, ner_ref, ptt_ref, ntt_ref, pos_out, ner_out):
    ptt = ptt_ref[...]
    ntt = ntt_ref[...]
    kp = jax.lax.broadcasted_iota(jnp.int32, (N_POS, 1), 0)
    kn = jax.lax.broadcasted_iota(jnp.int32, (N_NER, 1), 0)
    for s in range(_SBLK):
        prow = pos_ref[s:s + 1, :]
        oh = (kp == prow).astype(jnp.float32)
        res = jnp.dot(ptt, oh, preferred_element_type=jnp.float32)
        pos_out[:, s:s + 1, :] = res[:, None, :]
        nrow = ner_ref[s:s + 1, :]
        ohn = (kn == nrow).astype(jnp.float32)
        resn = jnp.dot(ntt, ohn, preferred_element_type=jnp.float32)
        ner_out[:, s:s + 1, :] = resn[:, None, :]


def kernel(words, masks, pos, ner, deprel, head, subj_pos, obj_pos, subj_type, obj_type,
           pos_table, ner_table):
    del words, deprel, subj_pos, obj_pos, subj_type, obj_type
    masks_t = masks.T
    head_t = head.T
    pos_t = pos.T
    ner_t = ner.T

    adj_t = pl.pallas_call(
        _mask_body,
        grid=(S // _RBLK,),
        in_specs=[
            pl.BlockSpec((S, B), lambda i: (0, 0)),
            pl.BlockSpec((S, B), lambda i: (0, 0)),
        ],
        out_specs=pl.BlockSpec((_RBLK, S, B), lambda i: (i, 0, 0)),
        out_shape=jax.ShapeDtypeStruct((S, S, B), jnp.float32),
    )(masks_t, head_t)

    adj = jnp.transpose(adj_t, (2, 0, 1))

    pos_et, ner_et = pl.pallas_call(
        _emb_body,
        grid=(S // _SBLK,),
        in_specs=[
            pl.BlockSpec((_SBLK, B), lambda i: (i, 0)),
            pl.BlockSpec((_SBLK, B), lambda i: (i, 0)),
            pl.BlockSpec((POS_DIM, N_POS), lambda i: (0, 0)),
            pl.BlockSpec((NER_DIM, N_NER), lambda i: (0, 0)),
        ],
        out_specs=[
            pl.BlockSpec((POS_DIM, _SBLK, B), lambda i: (0, i, 0)),
            pl.BlockSpec((NER_DIM, _SBLK, B), lambda i: (0, i, 0)),
        ],
        out_shape=[
            jax.ShapeDtypeStruct((POS_DIM, S, B), jnp.float32),
            jax.ShapeDtypeStruct((NER_DIM, S, B), jnp.float32),
        ],
    )(pos_t, ner_t, pos_table.T, ner_table.T)

    pos_embs = jnp.transpose(pos_et, (2, 1, 0))
    ner_embs = jnp.transpose(ner_et, (2, 1, 0))

    l = jnp.sum((masks == 0.0).astype(jnp.int32), axis=1)
    alen = jnp.arange(S)
    amask = alen[None, :] < l[:, None]
    pad_mask = jnp.broadcast_to((~amask)[:, None, :], (B, S, S))
    head_eff = jnp.where((head > 0) & amask, head, 0)
    dep_mask = head_eff[:, None, :] != (alen + 1)[None, :, None]
    seq_mask = jnp.broadcast_to(~(alen[None, None, :] <= alen[None, :, None]),
                                (B, S, S))

    return (pos_embs, ner_embs, dep_mask, pad_mask, seq_mask, adj)
